# baseline (device time: 8586 ns/iter reference)
import jax
import jax.numpy as jnp
from jax import lax
from jax.experimental import pallas as pl
from jax.experimental.pallas import tpu as pltpu

N_CHUNKS = 4


def kernel(x):
    m, n = x.shape[2], x.shape[3]
    half = m // 2
    rows = half // N_CHUNKS

    def body(x_hbm, out_ref, xf_ref, xb_ref, acc_ref, comm_ref,
             in_sems, send_sems, recv_sems):
        my_x = lax.axis_index("x")
        my_y = lax.axis_index("y")
        x_nbr = (1 - my_x, my_y)
        y_nbr = (my_x, 1 - my_y)
        partners = ((x_nbr, y_nbr), (y_nbr, x_nbr))

        def sl(p, c):
            return pl.ds(p * half + c * rows, rows)

        in_copies = {}
        for c in range(N_CHUNKS):
            for p in range(2):
                ic = pltpu.make_async_copy(
                    x_hbm.at[0, 0, sl(p, c)], xf_ref.at[sl(p, c)],
                    in_sems.at[p, c],
                )
                ic.start()
                in_copies[p, c] = ic

        barrier_sem = pltpu.get_barrier_semaphore()
        for nbr in (x_nbr, y_nbr):
            pl.semaphore_signal(
                barrier_sem, inc=1,
                device_id=nbr, device_id_type=pl.DeviceIdType.MESH,
            )
        pl.semaphore_wait(barrier_sem, 2)

        rdma1 = {}
        for c in range(N_CHUNKS):
            for p in range(2):
                in_copies[p, c].wait()
                s = sl(p, c)
                xb_ref[s, :] = xf_ref[s, :].astype(jnp.bfloat16)
                r = pltpu.make_async_remote_copy(
                    src_ref=xb_ref.at[s],
                    dst_ref=comm_ref.at[0, s],
                    send_sem=send_sems.at[0, p, c],
                    recv_sem=recv_sems.at[0, p, c],
                    device_id=partners[p][0],
                    device_id_type=pl.DeviceIdType.MESH,
                )
                r.start()
                rdma1[p, c] = r

        rdma2 = {}
        for c in range(N_CHUNKS):
            for p in range(2):
                rdma1[p, c].wait_recv()
                s = sl(p, c)
                acc_ref[s, :] = xb_ref[s, :] + comm_ref[0, s, :]
                r = pltpu.make_async_remote_copy(
                    src_ref=acc_ref.at[s],
                    dst_ref=comm_ref.at[1, s],
                    send_sem=send_sems.at[1, p, c],
                    recv_sem=recv_sems.at[1, p, c],
                    device_id=partners[p][1],
                    device_id_type=pl.DeviceIdType.MESH,
                )
                r.start()
                rdma2[p, c] = r

        for c in range(N_CHUNKS):
            for p in range(2):
                rdma2[p, c].wait_recv()
                s = sl(p, c)
                out_ref[s, :] = acc_ref[s, :] + comm_ref[1, s, :]

        for c in range(N_CHUNKS):
            for p in range(2):
                rdma1[p, c].wait_send()
                rdma2[p, c].wait_send()

    return pl.pallas_call(
        body,
        out_shape=jax.ShapeDtypeStruct((m, n), jnp.bfloat16),
        in_specs=[pl.BlockSpec(memory_space=pltpu.MemorySpace.HBM)],
        out_specs=pl.BlockSpec(memory_space=pltpu.VMEM),
        scratch_shapes=[
            pltpu.VMEM((m, n), jnp.float32),
            pltpu.VMEM((m, n), jnp.bfloat16),
            pltpu.VMEM((m, n), jnp.bfloat16),
            pltpu.VMEM((2, m, n), jnp.bfloat16),
            pltpu.SemaphoreType.DMA((2, N_CHUNKS)),
            pltpu.SemaphoreType.DMA((2, 2, N_CHUNKS)),
            pltpu.SemaphoreType.DMA((2, 2, N_CHUNKS)),
        ],
        compiler_params=pltpu.CompilerParams(collective_id=0),
    )(pltpu.with_memory_space_constraint(x, pltpu.MemorySpace.HBM))
